# baseline (speedup 1.0000x reference)
import jax
import jax.numpy as jnp
from jax.experimental import pallas as pl
from jax.experimental.pallas import tpu as pltpu

_N_E = 1024
_E_DIM = 256
_BETA = 0.25
_B = 16
_HW = 1024
_ROWS = _B * _HW


def _vq_body(z_ref, emb_ref, idx_ref, zq_ref, cnt_ref, loss_ref, perp_ref):
    i = pl.program_id(0)
    zt = z_ref[0]
    em = emb_ref[...]

    rn = jnp.sum(zt * zt, axis=0, keepdims=True)
    en = jnp.sum(em * em, axis=1, keepdims=True)
    mm = jax.lax.dot_general(
        em, zt, (((1,), (0,)), ((), ())),
        preferred_element_type=jnp.float32)
    d = (rn + en) - 2.0 * mm

    dmin = jnp.min(d, axis=0, keepdims=True)
    kio = jax.lax.broadcasted_iota(jnp.int32, d.shape, 0)
    idxv = jnp.min(jnp.where(d == dmin, kio, jnp.int32(2**30)), axis=0)
    idx_ref[0, 0, :] = idxv

    oh = (kio == idxv[None, :]).astype(jnp.float32)
    zqt = jax.lax.dot_general(
        em, oh, (((0,), (0,)), ((), ())),
        preferred_element_type=jnp.float32)
    zq_ref[0] = zt + (zqt - zt)

    @pl.when(i == 0)
    def _init():
        cnt_ref[...] = jnp.zeros_like(cnt_ref)
        loss_ref[...] = jnp.zeros_like(loss_ref)

    cnt_ref[0, :] += jnp.sum(oh, axis=1)
    loss_ref[...] += jnp.sum(dmin, keepdims=True)

    @pl.when(i == _B - 1)
    def _finalize():
        loss_ref[...] = loss_ref[...] * ((1.0 + _BETA) / (_ROWS * _E_DIM))
        cnt = cnt_ref[...]
        e_mean = cnt / jnp.sum(cnt, axis=1, keepdims=True)
        ent = jnp.sum(e_mean * jnp.log(e_mean + 1e-10), axis=1, keepdims=True)
        perp_ref[...] = jnp.exp(-ent)


def _vq_call(zr, emb):
    return pl.pallas_call(
        _vq_body,
        grid=(_B,),
        in_specs=[
            pl.BlockSpec((1, _E_DIM, _HW), lambda i: (i, 0, 0)),
            pl.BlockSpec((_N_E, _E_DIM), lambda i: (0, 0)),
        ],
        out_specs=[
            pl.BlockSpec((1, 1, _HW), lambda i: (i, 0, 0)),
            pl.BlockSpec((1, _E_DIM, _HW), lambda i: (i, 0, 0)),
            pl.BlockSpec((1, _N_E), lambda i: (0, 0)),
            pl.BlockSpec((1, 1), lambda i: (0, 0)),
            pl.BlockSpec((1, 1), lambda i: (0, 0)),
        ],
        out_shape=[
            jax.ShapeDtypeStruct((_B, 1, _HW), jnp.int32),
            jax.ShapeDtypeStruct((_B, _E_DIM, _HW), jnp.float32),
            jax.ShapeDtypeStruct((1, _N_E), jnp.float32),
            jax.ShapeDtypeStruct((1, 1), jnp.float32),
            jax.ShapeDtypeStruct((1, 1), jnp.float32),
        ],
        compiler_params=pltpu.CompilerParams(
            dimension_semantics=("arbitrary",),
        ),
    )(zr, emb)


def kernel(z, emb):
    b, c, h, w = z.shape
    zr = z.reshape(b, c, h * w)
    idx3, zq, cnt, loss, perp = _vq_call(zr, emb)
    enc_idx = idx3.reshape(b, h, w)
    z_q_out = zq.reshape(b, c, h, w)
    return (loss.reshape(()), z_q_out, perp.reshape(()),
            cnt.reshape(_N_E), enc_idx)
